# baseline (speedup 1.0000x reference)
import jax
import jax.numpy as jnp
from jax import lax
from jax.experimental import pallas as pl
from jax.experimental.pallas import tpu as pltpu

_BB = 2
_NBUF = 32
_LA = 16


def kernel(x, removed_electrodes):
    B, C, T, E = x.shape
    xt = jnp.transpose(x, (0, 3, 1, 2))
    NB = B // _BB
    TOT = E * NB
    rem = removed_electrodes.astype(jnp.int32)
    zeros = jnp.zeros((_BB, C, T), x.dtype)

    def body(rem_ref, z_ref, x_hbm, o_hbm, buf, in_sems, out_sems):
        e = pl.program_id(0)
        j = pl.program_id(1)
        s = e * NB + j

        def keep_of(t):
            e_t = t // NB
            k = jnp.int32(1)
            for i in range(rem_ref.shape[0]):
                k = k * (e_t + 1 != rem_ref[i]).astype(jnp.int32)
            return k

        def slices(t):
            e_t = t // NB
            j_t = lax.rem(t, NB)
            return pl.ds(j_t * _BB, _BB), e_t

        def issue_in(t):
            @pl.when(keep_of(t) == 1)
            def _():
                bsl, e_t = slices(t)
                slot = lax.rem(t, _NBUF)
                pltpu.make_async_copy(
                    x_hbm.at[bsl, e_t], buf.at[slot], in_sems.at[slot]
                ).start()

        def wait_in(t):
            slot = lax.rem(t, _NBUF)
            pltpu.make_async_copy(
                x_hbm.at[pl.ds(0, _BB), 0], buf.at[slot], in_sems.at[slot]
            ).wait()

        def issue_out(t):
            bsl, e_t = slices(t)
            slot = lax.rem(t, _NBUF)
            kp = keep_of(t)

            @pl.when(kp == 1)
            def _():
                pltpu.make_async_copy(
                    buf.at[slot], o_hbm.at[bsl, e_t], out_sems.at[slot]
                ).start()

            @pl.when(kp == 0)
            def _():
                pltpu.make_async_copy(
                    z_ref, o_hbm.at[bsl, e_t], out_sems.at[slot]
                ).start()

        def wait_out(t):
            bsl, e_t = slices(t)
            pltpu.make_async_copy(
                z_ref, o_hbm.at[bsl, e_t], out_sems.at[lax.rem(t, _NBUF)]
            ).wait()

        @pl.when(s == 0)
        def _():
            for t0 in range(_LA):
                issue_in(jnp.int32(t0))

        @pl.when(s >= _LA)
        def _():
            wait_out(s - _LA)

        @pl.when(s + _LA < TOT)
        def _():
            issue_in(s + _LA)

        @pl.when(keep_of(s) == 1)
        def _():
            wait_in(s)

        issue_out(s)

        @pl.when(s == TOT - 1)
        def _():
            for d in range(_LA):
                wait_out(s - d)

    out_t = pl.pallas_call(
        body,
        grid=(E, NB),
        in_specs=[
            pl.BlockSpec(memory_space=pltpu.SMEM),
            pl.BlockSpec(memory_space=pltpu.VMEM),
            pl.BlockSpec(memory_space=pl.ANY),
        ],
        out_specs=pl.BlockSpec(memory_space=pl.ANY),
        out_shape=jax.ShapeDtypeStruct((B, E, C, T), x.dtype),
        scratch_shapes=[
            pltpu.VMEM((_NBUF, _BB, C, T), x.dtype),
            pltpu.SemaphoreType.DMA((_NBUF,)),
            pltpu.SemaphoreType.DMA((_NBUF,)),
        ],
    )(rem, zeros, xt)
    return jnp.transpose(out_t, (0, 2, 3, 1))


# speedup vs baseline: 1.0281x; 1.0281x over previous
import jax
import jax.numpy as jnp
from jax import lax
from jax.experimental import pallas as pl
from jax.experimental.pallas import tpu as pltpu

_BB = 8
_NBUF = 12
_LA = 6


def kernel(x, removed_electrodes):
    B, C, T, E = x.shape
    xt = jnp.transpose(x, (0, 3, 1, 2))
    NB = B // _BB
    TOT = E * NB
    rem = removed_electrodes.astype(jnp.int32)
    zeros = jnp.zeros((_BB, C, T), x.dtype)

    def body(rem_ref, z_ref, x_hbm, o_hbm, buf, in_sems, out_sems):
        e = pl.program_id(0)
        j = pl.program_id(1)
        s = e * NB + j

        def keep_of(t):
            e_t = t // NB
            k = jnp.int32(1)
            for i in range(rem_ref.shape[0]):
                k = k * (e_t + 1 != rem_ref[i]).astype(jnp.int32)
            return k

        def slices(t):
            e_t = t // NB
            j_t = lax.rem(t, NB)
            return pl.ds(j_t * _BB, _BB), e_t

        def issue_in(t):
            @pl.when(keep_of(t) == 1)
            def _():
                bsl, e_t = slices(t)
                slot = lax.rem(t, _NBUF)
                pltpu.make_async_copy(
                    x_hbm.at[bsl, e_t], buf.at[slot], in_sems.at[slot]
                ).start()

        def wait_in(t):
            slot = lax.rem(t, _NBUF)
            pltpu.make_async_copy(
                x_hbm.at[pl.ds(0, _BB), 0], buf.at[slot], in_sems.at[slot]
            ).wait()

        def issue_out(t):
            bsl, e_t = slices(t)
            slot = lax.rem(t, _NBUF)
            kp = keep_of(t)

            @pl.when(kp == 1)
            def _():
                pltpu.make_async_copy(
                    buf.at[slot], o_hbm.at[bsl, e_t], out_sems.at[slot]
                ).start()

            @pl.when(kp == 0)
            def _():
                pltpu.make_async_copy(
                    z_ref, o_hbm.at[bsl, e_t], out_sems.at[slot]
                ).start()

        def wait_out(t):
            bsl, e_t = slices(t)
            pltpu.make_async_copy(
                z_ref, o_hbm.at[bsl, e_t], out_sems.at[lax.rem(t, _NBUF)]
            ).wait()

        @pl.when(s == 0)
        def _():
            for t0 in range(_LA):
                issue_in(jnp.int32(t0))

        @pl.when(s >= _LA)
        def _():
            wait_out(s - _LA)

        @pl.when(s + _LA < TOT)
        def _():
            issue_in(s + _LA)

        @pl.when(keep_of(s) == 1)
        def _():
            wait_in(s)

        issue_out(s)

        @pl.when(s == TOT - 1)
        def _():
            for d in range(_LA):
                wait_out(s - d)

    out_t = pl.pallas_call(
        body,
        grid=(E, NB),
        in_specs=[
            pl.BlockSpec(memory_space=pltpu.SMEM),
            pl.BlockSpec(memory_space=pltpu.VMEM),
            pl.BlockSpec(memory_space=pl.ANY),
        ],
        out_specs=pl.BlockSpec(memory_space=pl.ANY),
        out_shape=jax.ShapeDtypeStruct((B, E, C, T), x.dtype),
        scratch_shapes=[
            pltpu.VMEM((_NBUF, _BB, C, T), x.dtype),
            pltpu.SemaphoreType.DMA((_NBUF,)),
            pltpu.SemaphoreType.DMA((_NBUF,)),
        ],
    )(rem, zeros, xt)
    return jnp.transpose(out_t, (0, 2, 3, 1))
